# baseline (device time: 58115 ns/iter reference)
import jax
import jax.numpy as jnp
from jax import lax
from jax.experimental import pallas as pl
from jax.experimental.pallas import tpu as pltpu

N_DEV = 4


def kernel(A, B):
    m, k = A.shape
    _, n = B.shape
    ch = m // N_DEV
    half = n // 2

    def body(a_hbm, b_hbm, out_hbm, a_vm, b_vm, a_bf, b_bf, acc_bf, out_vm,
             rs1_comm, rs2_comm, ag1_comm, ag2_comm,
             a_sems, b_sems, own_sem, wb_sems,
             rs1_send, rs1_recv, rs2_send, rs2_recv,
             ag1_send, ag1_recv, ag2_send, ag2_recv):
        my = lax.axis_index("i")
        px = my ^ 3
        py = my ^ 1
        p1 = (px, py)
        p2 = (py, px)

        def rows(c):
            return pl.ds(c * ch, ch)

        def hcols(d):
            return pl.ds(d * half, half)

        def cp_a(c):
            return pltpu.make_async_copy(
                a_hbm.at[rows(c), :], a_vm.at[rows(c), :], a_sems.at[c]
            )

        def cp_b(d):
            return pltpu.make_async_copy(
                b_hbm.at[:, hcols(d)], b_vm.at[:, hcols(d)], b_sems.at[d]
            )

        cp_a(my ^ 2).start()
        cp_b(0).start()
        cp_b(1).start()
        cp_a(my ^ 3).start()
        cp_a(my ^ 1).start()
        cp_a(my).start()

        barrier = pltpu.get_barrier_semaphore()
        for nbr in (px, py):
            pl.semaphore_signal(
                barrier, inc=1,
                device_id=(nbr,), device_id_type=pl.DeviceIdType.MESH,
            )
        pl.semaphore_wait(barrier, 2)

        def cva(c):
            cp_a(c).wait()
            a_bf[rows(c), :] = a_vm[rows(c), :].astype(jnp.bfloat16)

        def cvb(d):
            cp_b(d).wait()
            b_bf[:, hcols(d)] = b_vm[:, hcols(d)].astype(jnp.bfloat16)

        def mmh(c, d):
            acc_bf[rows(c), hcols(d)] = jnp.dot(
                a_bf[rows(c), :], b_bf[:, hcols(d)],
                preferred_element_type=jnp.float32,
            ).astype(jnp.bfloat16)

        rs1_send_chunk = ((my ^ 2, my ^ 3), (my ^ 2, my ^ 1))
        rs1_recv_chunk = ((my ^ 1, my), (my ^ 3, my))

        def rs1_rdma(d, j):
            return pltpu.make_async_remote_copy(
                src_ref=acc_bf.at[rows(rs1_send_chunk[d][j]), hcols(d)],
                dst_ref=rs1_comm.at[d, j],
                send_sem=rs1_send.at[d, j],
                recv_sem=rs1_recv.at[d, j],
                device_id=(p1[d],),
                device_id_type=pl.DeviceIdType.MESH,
            )

        rs2_send_chunk = (my ^ 1, my ^ 3)

        def rs2_rdma(d):
            return pltpu.make_async_remote_copy(
                src_ref=acc_bf.at[rows(rs2_send_chunk[d]), hcols(d)],
                dst_ref=rs2_comm.at[d],
                send_sem=rs2_send.at[d],
                recv_sem=rs2_recv.at[d],
                device_id=(p2[d],),
                device_id_type=pl.DeviceIdType.MESH,
            )

        def ag1_rdma(d):
            return pltpu.make_async_remote_copy(
                src_ref=acc_bf.at[rows(my), hcols(d)],
                dst_ref=ag1_comm.at[d],
                send_sem=ag1_send.at[d],
                recv_sem=ag1_recv.at[d],
                device_id=(p2[d],),
                device_id_type=pl.DeviceIdType.MESH,
            )

        ag2_recv_chunk = ((my ^ 3, my ^ 2), (my ^ 1, my ^ 2))

        def ag2_rdma(d, j):
            src = acc_bf.at[rows(my), hcols(d)] if j == 0 else ag1_comm.at[d]
            return pltpu.make_async_remote_copy(
                src_ref=src,
                dst_ref=ag2_comm.at[d, j],
                send_sem=ag2_send.at[d, j],
                recv_sem=ag2_recv.at[d, j],
                device_id=(p1[d],),
                device_id_type=pl.DeviceIdType.MESH,
            )

        def store_wb(c, d, comm, sem):
            out_vm[rows(c), hcols(d)] = comm.astype(jnp.float32)
            return pltpu.make_async_copy(
                out_vm.at[rows(c), hcols(d)], out_hbm.at[rows(c), hcols(d)],
                sem,
            )

        cva(my ^ 2)
        cvb(0)
        mmh(my ^ 2, 0)
        rs1_rdma(0, 0).start()
        cvb(1)
        mmh(my ^ 2, 1)
        rs1_rdma(1, 0).start()
        cva(my ^ 3)
        mmh(my ^ 3, 0)
        rs1_rdma(0, 1).start()
        cva(my ^ 1)
        mmh(my ^ 1, 1)
        rs1_rdma(1, 1).start()
        mmh(my ^ 1, 0)
        mmh(my ^ 3, 1)
        cva(my)
        mmh(my, 0)
        mmh(my, 1)

        for d in (0, 1):
            r = rs1_rdma(d, 0)
            r.wait_recv()
            c = rs1_recv_chunk[d][0]
            acc_bf[rows(c), hcols(d)] = (
                acc_bf[rows(c), hcols(d)] + rs1_comm[d, 0]
            )
            rs2_rdma(d).start()
        for d in (0, 1):
            r = rs1_rdma(d, 1)
            r.wait_recv()
            acc_bf[rows(my), hcols(d)] = (
                acc_bf[rows(my), hcols(d)] + rs1_comm[d, 1]
            )
        for d in (0, 1):
            r = rs2_rdma(d)
            r.wait_recv()
            acc_bf[rows(my), hcols(d)] = jnp.maximum(
                acc_bf[rows(my), hcols(d)] + rs2_comm[d], 0.0
            )
            ag1_rdma(d).start()
            ag2_rdma(d, 0).start()
        out_vm[rows(my), :] = acc_bf[rows(my), :].astype(jnp.float32)
        pltpu.make_async_copy(
            out_vm.at[rows(my), :], out_hbm.at[rows(my), :], own_sem.at[0]
        ).start()

        wbs = []
        for d in (0, 1):
            r = ag1_rdma(d)
            r.wait_recv()
            ag2_rdma(d, 1).start()
            c = my ^ 1 if d == 0 else my ^ 3
            wb = store_wb(c, d, ag1_comm[d], wb_sems.at[d, 0])
            wb.start()
            wbs.append(wb)
        for j in (0, 1):
            for d in (0, 1):
                r = ag2_rdma(d, j)
                r.wait_recv()
                wb = store_wb(
                    ag2_recv_chunk[d][j], d, ag2_comm[d, j],
                    wb_sems.at[d, 1 + j],
                )
                wb.start()
                wbs.append(wb)

        for d in (0, 1):
            rs1_rdma(d, 0).wait_send()
            rs1_rdma(d, 1).wait_send()
            rs2_rdma(d).wait_send()
            ag1_rdma(d).wait_send()
            ag2_rdma(d, 0).wait_send()
            ag2_rdma(d, 1).wait_send()
        for wb in wbs:
            wb.wait()
        pltpu.make_async_copy(
            out_vm.at[rows(my), :], out_hbm.at[rows(my), :], own_sem.at[0]
        ).wait()

    return pl.pallas_call(
        body,
        out_shape=jax.ShapeDtypeStruct((m, n), jnp.float32),
        in_specs=[
            pl.BlockSpec(memory_space=pltpu.MemorySpace.HBM),
            pl.BlockSpec(memory_space=pltpu.MemorySpace.HBM),
        ],
        out_specs=pl.BlockSpec(memory_space=pltpu.MemorySpace.HBM),
        scratch_shapes=[
            pltpu.VMEM((m, k), jnp.float32),
            pltpu.VMEM((k, n), jnp.float32),
            pltpu.VMEM((m, k), jnp.bfloat16),
            pltpu.VMEM((k, n), jnp.bfloat16),
            pltpu.VMEM((m, n), jnp.bfloat16),
            pltpu.VMEM((m, n), jnp.float32),
            pltpu.VMEM((2, 2, ch, half), jnp.bfloat16),
            pltpu.VMEM((2, ch, half), jnp.bfloat16),
            pltpu.VMEM((2, ch, half), jnp.bfloat16),
            pltpu.VMEM((2, 2, ch, half), jnp.bfloat16),
            pltpu.SemaphoreType.DMA((N_DEV,)),
            pltpu.SemaphoreType.DMA((2,)),
            pltpu.SemaphoreType.DMA((1,)),
            pltpu.SemaphoreType.DMA((2, 3)),
            pltpu.SemaphoreType.DMA((2, 2)),
            pltpu.SemaphoreType.DMA((2, 2)),
            pltpu.SemaphoreType.DMA((2,)),
            pltpu.SemaphoreType.DMA((2,)),
            pltpu.SemaphoreType.DMA((2,)),
            pltpu.SemaphoreType.DMA((2,)),
            pltpu.SemaphoreType.DMA((2, 2)),
            pltpu.SemaphoreType.DMA((2, 2)),
        ],
        compiler_params=pltpu.CompilerParams(
            collective_id=0, vmem_limit_bytes=64 * 1024 * 1024
        ),
    )(A, B)
